# baseline (speedup 1.0000x reference)
import functools

import jax
import jax.numpy as jnp
from jax import lax
from jax.experimental import pallas as pl
from jax.experimental.pallas import tpu as pltpu
from jax.experimental.pallas import tpu_sc as plsc

NN = 10000
EE = 320000
IN_C = 128
HID = 64
OUT_C = 128

NC = 2
NS = 16
NW = NC * NS

N2 = 10240
E2 = 327680
IDX_ROWS = E2 // 128
ROWS_PER_W = IDX_ROWS // NW
SUP = 4
NSUP = ROWS_PER_W // SUP
RPT = N2 // NS
CR = 80
NCR = RPT // CR


def _sc_mesh():
    return plsc.VectorSubcoreMesh(core_axis_name="c", subcore_axis_name="s",
                                  num_cores=NC, num_subcores=NS)


def _agg_phase(table, src2, dst2, acc, src_v, dst_v, rows_v, gsem, ssem,
               w, hbm_dummy, dacc=None, ones_v=None, ones16=None):
    base = w * ROWS_PER_W

    def _scatter_waits(j):
        pltpu.make_async_copy(hbm_dummy,
                              rows_v.at[pl.ds(j * 128, 128)],
                              ssem.at[j]).wait()
        if dacc is not None:
            pltpu.make_async_copy(ones16, ones_v, ssem.at[j]).wait()

    def chunk(i, carry):
        ro = base + i * SUP
        par = lax.rem(i, 2)
        pltpu.sync_copy(src2.at[pl.ds(ro, SUP)], src_v.at[par])
        pltpu.sync_copy(dst2.at[pl.ds(ro, SUP)], dst_v.at[par])

        @pl.when(i > 0)
        def _():
            for j in range(SUP):
                _scatter_waits(j)

        gcs = [pltpu.async_copy(table.at[src_v.at[par, j]],
                                rows_v.at[pl.ds(j * 128, 128)],
                                gsem.at[j])
               for j in range(SUP)]
        for j in range(SUP):
            gcs[j].wait()
            pltpu.async_copy(rows_v.at[pl.ds(j * 128, 128)],
                             acc.at[dst_v.at[par, j]], ssem.at[j], add=True)
            if dacc is not None:
                pltpu.async_copy(ones_v, dacc.at[dst_v.at[par, j]],
                                 ssem.at[j], add=True)
        return carry

    lax.fori_loop(0, NSUP, chunk, 0)
    for j in range(SUP):
        _scatter_waits(j)


def _zero_acc_slice(rows_v, acc, r0):
    zv = jnp.zeros((16,), jnp.float32)
    half = RPT // 2

    def zrow(r, carry):
        for j in range(HID // 16):
            rows_v[r, pl.ds(j * 16, 16)] = zv
        return carry

    lax.fori_loop(0, half, zrow, 0)
    pltpu.sync_copy(rows_v.at[pl.ds(0, half)], acc.at[pl.ds(r0, half)])
    pltpu.sync_copy(rows_v.at[pl.ds(0, half)],
                    acc.at[pl.ds(r0 + half, half)])


def _combine_chunk(p, degp, z, b1_v, relu, rows_v, d0, d1, cr0):
    pltpu.sync_copy(p.at[0, pl.ds(cr0, CR)], rows_v.at[pl.ds(0, CR)])
    pltpu.sync_copy(p.at[1, pl.ds(cr0, CR)], rows_v.at[pl.ds(CR, CR)])
    if z is not None:
        pltpu.sync_copy(z.at[pl.ds(cr0, CR)], rows_v.at[pl.ds(2 * CR, CR)])
    pltpu.sync_copy(degp.at[0, pl.ds(cr0, CR)], d0)
    pltpu.sync_copy(degp.at[1, pl.ds(cr0, CR)], d1)

    def row(r, carry):
        dinv = 1.0 / jnp.maximum(d0[r, :] + d1[r, :], 1.0)
        for j in range(HID // 16):
            sl = pl.ds(j * 16, 16)
            t = (rows_v[r, sl] + rows_v[CR + r, sl]) * dinv
            if z is not None:
                t = t + rows_v[2 * CR + r, sl]
            if relu:
                t = jnp.maximum(t + b1_v[sl], 0.0)
            rows_v[3 * CR + r, sl] = t
        return carry

    lax.fori_loop(0, CR, row, 0)


def _make_sc_app(first, z_add, relu, table_out):
    out_type = [jax.ShapeDtypeStruct((NC, N2, HID), jnp.float32)]
    if first:
        out_type.append(jax.ShapeDtypeStruct((NC, N2, 16), jnp.float32))
    if table_out:
        out_type.append(jax.ShapeDtypeStruct((N2, HID), jnp.float32))

    scratch = [
        pltpu.VMEM_SHARED((N2, HID), jnp.float32),
        pltpu.VMEM_SHARED((N2, HID), jnp.float32),
        pltpu.VMEM((2, SUP, 128), jnp.int32),
        pltpu.VMEM((2, SUP, 128), jnp.int32),
        pltpu.VMEM((SUP * 128, HID), jnp.float32),
        pltpu.SemaphoreType.DMA((SUP,)),
        pltpu.SemaphoreType.DMA((SUP,)),
    ]
    if first:
        scratch += [pltpu.VMEM_SHARED((N2, 16), jnp.float32),
                    pltpu.VMEM((128, 16), jnp.float32)]
    else:
        scratch += [pltpu.VMEM((CR, 16), jnp.float32),
                    pltpu.VMEM((CR, 16), jnp.float32),
                    pltpu.VMEM((HID,), jnp.float32)]

    def body(*refs):
        if first:
            (tblin, src2, dst2, zeros16, ones16, pout, dout,
             table, acc, src_v, dst_v, rows_v, gsem, ssem,
             dacc, ones_v) = refs
        else:
            p_in, src2, dst2, degp = refs[:4]
            i = 4
            z = b1 = None
            if z_add:
                z = refs[i]
                i += 1
            if relu:
                b1 = refs[i]
                i += 1
            pout = refs[i]
            i += 1
            tout = None
            if table_out:
                tout = refs[i]
                i += 1
            (table, acc, src_v, dst_v, rows_v, gsem, ssem,
             d0, d1, b1_v) = refs[i:]

        c = lax.axis_index("c")
        s = lax.axis_index("s")
        w = s * NC + c
        r0 = s * RPT

        if first:
            pltpu.sync_copy(tblin.at[pl.ds(r0, RPT)], table.at[pl.ds(r0, RPT)])
            pltpu.sync_copy(zeros16.at[pl.ds(r0, RPT)],
                            dacc.at[pl.ds(r0, RPT)])
            pltpu.sync_copy(ones16, ones_v)
            hbm_dummy = tblin.at[pl.ds(0, 128)]
        else:
            if relu:
                pltpu.sync_copy(b1, b1_v)
            for k in range(NCR):
                cr0 = r0 + k * CR
                _combine_chunk(p_in, degp, z, b1_v, relu,
                               rows_v, d0, d1, cr0)
                pltpu.sync_copy(rows_v.at[pl.ds(3 * CR, CR)],
                                table.at[pl.ds(cr0, CR)])
                if table_out:
                    @pl.when(c == 0)
                    def _():
                        pltpu.sync_copy(rows_v.at[pl.ds(3 * CR, CR)],
                                        tout.at[pl.ds(cr0, CR)])
            hbm_dummy = p_in.at[0, pl.ds(0, 128)]

        _zero_acc_slice(rows_v, acc, r0)
        plsc.subcore_barrier()

        if first:
            _agg_phase(table, src2, dst2, acc, src_v, dst_v, rows_v, gsem,
                       ssem, w, hbm_dummy, dacc=dacc, ones_v=ones_v,
                       ones16=ones16)
        else:
            _agg_phase(table, src2, dst2, acc, src_v, dst_v, rows_v, gsem,
                       ssem, w, hbm_dummy)
        plsc.subcore_barrier()

        pltpu.sync_copy(acc.at[pl.ds(r0, RPT)], pout.at[c, pl.ds(r0, RPT)])
        if first:
            pltpu.sync_copy(dacc.at[pl.ds(r0, RPT)],
                            dout.at[c, pl.ds(r0, RPT)])

    return pl.kernel(
        body,
        out_type=tuple(out_type) if len(out_type) > 1 else out_type[0],
        mesh=_sc_mesh(),
        compiler_params=pltpu.CompilerParams(use_tc_tiling_on_sc=False),
        scratch_types=tuple(scratch),
    )


_app_first = _make_sc_app(first=True, z_add=False, relu=False, table_out=False)
_app_z = _make_sc_app(first=False, z_add=True, relu=False, table_out=False)
_app_relu = _make_sc_app(first=False, z_add=True, relu=True, table_out=True)
_app_plain = _make_sc_app(first=False, z_add=False, relu=False, table_out=True)


def _zmm(x2, W1):
    def body(x_ref, w_ref, z_ref):
        for k in range(4):
            z_ref[k] = jnp.dot(x_ref[...], w_ref[k],
                               preferred_element_type=jnp.float32)

    return pl.pallas_call(
        body,
        out_shape=jax.ShapeDtypeStruct((4, N2, HID), jnp.float32),
    )(x2, W1)


def _fin(h, a1, a2, p, degp, W2, b2):
    def body(h_ref, a1_ref, a2_ref, p_ref, dp_ref, w_ref, b_ref, y_ref):
        dinv = 1.0 / jnp.maximum(dp_ref[0, :, :1] + dp_ref[1, :, :1], 1.0)
        a3 = (p_ref[0] + p_ref[1]) * dinv
        acc = jnp.dot(h_ref[...], w_ref[0], preferred_element_type=jnp.float32)
        acc += jnp.dot(a1_ref[...], w_ref[1], preferred_element_type=jnp.float32)
        acc += jnp.dot(a2_ref[...], w_ref[2], preferred_element_type=jnp.float32)
        acc += jnp.dot(a3, w_ref[3], preferred_element_type=jnp.float32)
        y_ref[...] = acc + b_ref[...]

    return pl.pallas_call(
        body, out_shape=jax.ShapeDtypeStruct((N2, OUT_C), jnp.float32),
    )(h, a1, a2, p, degp, W2, b2)


def kernel(x, edge_index, W1, b1, W2, b2):
    src = edge_index[0]
    dst = edge_index[1]
    pad = E2 - EE
    pidx = lax.iota(jnp.int32, pad)
    src2 = jnp.concatenate([src, pidx % 128]).reshape(IDX_ROWS, 128)
    dst2 = jnp.concatenate([dst, NN + (pidx % 8)]).reshape(IDX_ROWS, 128)
    x2 = jnp.pad(x, ((0, N2 - NN), (0, 0)))
    zeros16 = jnp.zeros((N2, 16), jnp.float32)
    ones16 = jnp.ones((128, 16), jnp.float32)

    z = _zmm(x2, W1)
    p, degp = _app_first(z[3], src2, dst2, zeros16, ones16)
    p = _app_z(p, src2, dst2, degp, z[2])
    p = _app_z(p, src2, dst2, degp, z[1])
    p, h = _app_relu(p, src2, dst2, degp, z[0], b1)
    p, a1 = _app_plain(p, src2, dst2, degp)
    p, a2 = _app_plain(p, src2, dst2, degp)
    y2 = _fin(h, a1, a2, p, degp, W2, b2)
    return y2[:NN]
